# Initial kernel scaffold; baseline (speedup 1.0000x reference)
import jax
import jax.numpy as jnp
from jax.experimental import pallas as pl


def kernel(z, a):
    raise NotImplementedError("write your pallas kernel here")


# speedup vs baseline: 3.9864x; 3.9864x over previous
import functools

import jax
import jax.numpy as jnp
from jax.experimental import pallas as pl

_ATTR_INDEX = 8
_BLOCK_B = 4096


def _body(z_ref, a_ref, o_ref):
    idx = z_ref[:, _ATTR_INDEX].astype(jnp.int32)
    c = a_ref.shape[1]
    iota = jax.lax.broadcasted_iota(jnp.int32, (z_ref.shape[0], c), 1)
    mask = iota == idx[:, None]
    o_ref[:] = jnp.sum(jnp.where(mask, a_ref[:], 0.0), axis=1) * 0.999


@jax.jit
def kernel(z, a):
    b, _ = a.shape
    grid = (b // _BLOCK_B,)
    return pl.pallas_call(
        _body,
        grid=grid,
        in_specs=[
            pl.BlockSpec((_BLOCK_B, z.shape[1]), lambda i: (i, 0)),
            pl.BlockSpec((_BLOCK_B, a.shape[1]), lambda i: (i, 0)),
        ],
        out_specs=pl.BlockSpec((_BLOCK_B,), lambda i: (i,)),
        out_shape=jax.ShapeDtypeStruct((b,), jnp.float32),
    )(z, a)
